# Initial kernel scaffold; baseline (speedup 1.0000x reference)
import jax
import jax.numpy as jnp
from jax.experimental import pallas as pl


def kernel(x, edge_index, batch, global_features, conv_params, gmlp_params, reg_params):
    raise NotImplementedError("write your pallas kernel here")


# speedup vs baseline: 12.0238x; 12.0238x over previous
import functools

import jax
import jax.numpy as jnp
import numpy as np
from jax import lax
from jax.experimental import pallas as pl
from jax.experimental.pallas import tpu as pltpu
from jax.experimental.pallas import tpu_sc as plsc

N = 10000
E = 320000
HEADS = 8
CH = 32
HID = 256
NC = 2
NS = 16
NW = NC * NS

EPT = E // NW
CEG = 40
NCH_G = EPT // CEG
CE = 80
EPS = E // NS
NCH_S = EPS // CE
NPAD = 10240
RPP = NPAD // NS

_mesh = plsc.VectorSubcoreMesh(core_axis_name="c", subcore_axis_name="s",
                               num_cores=NC, num_subcores=NS)


def _qkvs_body(x_ref, w_ref, b_ref, q_ref, k_ref, v_ref, s_ref):
    y = jnp.dot(x_ref[...], w_ref[...],
                preferred_element_type=jnp.float32) + b_ref[...]
    q_ref[...] = y[:, 0:256]
    k_ref[...] = y[:, 256:512]
    v_ref[...] = y[:, 512:768]
    s_ref[...] = y[:, 768:1024]


def _qkvs(h, W4, b4):
    d = h.shape[1]
    out = jax.ShapeDtypeStruct((N, HID), jnp.float32)
    return pl.pallas_call(
        _qkvs_body,
        grid=(N // 400,),
        in_specs=[
            pl.BlockSpec((400, d), lambda i: (i, 0)),
            pl.BlockSpec((d, 1024), lambda i: (0, 0)),
            pl.BlockSpec((1, 1024), lambda i: (0, 0)),
        ],
        out_specs=[pl.BlockSpec((400, HID), lambda i: (i, 0))] * 4,
        out_shape=[out] * 4,
    )(h, W4, b4)


@functools.partial(
    pl.kernel,
    mesh=_mesh,
    out_type=[jax.ShapeDtypeStruct((E, HID), jnp.float32)] * 3,
    scratch_types=[
        pltpu.VMEM((CEG,), jnp.int32),
        pltpu.VMEM((CEG,), jnp.int32),
        pltpu.VMEM((CEG, HID), jnp.float32),
        pltpu.SemaphoreType.DMA,
    ],
)
def _gather3(q_hbm, k_hbm, v_hbm, dst_hbm, src_hbm,
             qd_hbm, ks_hbm, vs_hbm, dst_v, src_v, rows_v, sem):
    wid = lax.axis_index("s") * NC + lax.axis_index("c")
    base = wid * EPT

    def gchunk(i, _):
        off = base + i * CEG
        pltpu.sync_copy(dst_hbm.at[pl.ds(off, CEG)], dst_v)
        pltpu.sync_copy(src_hbm.at[pl.ds(off, CEG)], src_v)
        pltpu.async_copy(q_hbm.at[dst_v], rows_v, sem).wait()
        pltpu.sync_copy(rows_v, qd_hbm.at[pl.ds(off, CEG), :])
        pltpu.async_copy(k_hbm.at[src_v], rows_v, sem).wait()
        pltpu.sync_copy(rows_v, ks_hbm.at[pl.ds(off, CEG), :])
        pltpu.async_copy(v_hbm.at[src_v], rows_v, sem).wait()
        pltpu.sync_copy(rows_v, vs_hbm.at[pl.ds(off, CEG), :])
        return 0

    lax.fori_loop(0, NCH_G, gchunk, 0)


EB = 2000


def _alpha_body(qd_ref, ks_ref, alpha_ref, gmax_ref):
    i = pl.program_id(0)
    s = qd_ref[...] * ks_ref[...]
    a = jnp.concatenate(
        [jnp.sum(s[:, h * CH:(h + 1) * CH], axis=1, keepdims=True)
         for h in range(HEADS)], axis=1)
    a = a * (1.0 / np.sqrt(CH))
    alpha_ref[...] = a
    m = jnp.max(a, axis=0, keepdims=True)

    @pl.when(i == 0)
    def _():
        gmax_ref[...] = m

    @pl.when(i > 0)
    def _():
        gmax_ref[...] = jnp.maximum(gmax_ref[...], m)


def _alpha(qd, ks):
    return pl.pallas_call(
        _alpha_body,
        grid=(E // EB,),
        in_specs=[
            pl.BlockSpec((EB, HID), lambda i: (i, 0)),
            pl.BlockSpec((EB, HID), lambda i: (i, 0)),
        ],
        out_specs=[
            pl.BlockSpec((EB, HEADS), lambda i: (i, 0)),
            pl.BlockSpec((1, HEADS), lambda i: (0, 0)),
        ],
        out_shape=[
            jax.ShapeDtypeStruct((E, HEADS), jnp.float32),
            jax.ShapeDtypeStruct((1, HEADS), jnp.float32),
        ],
    )(qd, ks)


def _msg_body(alpha_ref, gmax_ref, vs_ref, ex_ref, msg_ref):
    ex = jnp.exp(alpha_ref[...] - gmax_ref[...])
    exb = jnp.concatenate(
        [jnp.broadcast_to(ex[:, h:h + 1], (EB, CH)) for h in range(HEADS)],
        axis=1)
    ex_ref[0, :, :] = exb[:, 0:128]
    ex_ref[1, :, :] = exb[:, 128:256]
    m = exb * vs_ref[...]
    msg_ref[0, :, :] = m[:, 0:128]
    msg_ref[1, :, :] = m[:, 128:256]


def _msg(alpha, gmax, vs):
    return pl.pallas_call(
        _msg_body,
        grid=(E // EB,),
        in_specs=[
            pl.BlockSpec((EB, HEADS), lambda i: (i, 0)),
            pl.BlockSpec((1, HEADS), lambda i: (0, 0)),
            pl.BlockSpec((EB, HID), lambda i: (i, 0)),
        ],
        out_specs=[
            pl.BlockSpec((NC, EB, 128), lambda i: (0, i, 0)),
            pl.BlockSpec((NC, EB, 128), lambda i: (0, i, 0)),
        ],
        out_shape=[
            jax.ShapeDtypeStruct((NC, E, 128), jnp.float32),
            jax.ShapeDtypeStruct((NC, E, 128), jnp.float32),
        ],
    )(alpha, gmax, vs)


@functools.partial(
    pl.kernel,
    mesh=_mesh,
    out_type=jax.ShapeDtypeStruct((NC, NPAD, 128), jnp.float32),
    scratch_types=[
        pltpu.VMEM((CE,), jnp.int32),
        pltpu.VMEM((CE, 128), jnp.float32),
        pltpu.VMEM_SHARED((NPAD, 128), jnp.float32),
    ],
)
def _scat128(data_hbm, dst_hbm, zrow_hbm, acc_out, dst_v, row_v, acc_sh):
    c = lax.axis_index("c")
    s = lax.axis_index("s")
    rb = s * RPP
    base = s * EPS

    pltpu.sync_copy(zrow_hbm, row_v)

    def zc(i, _):
        pltpu.sync_copy(row_v, acc_sh.at[pl.ds(rb + i * CE, CE), :])
        return 0

    lax.fori_loop(0, RPP // CE, zc, 0)
    plsc.subcore_barrier()

    def chunk(i, _):
        off = base + i * CE
        pltpu.sync_copy(dst_hbm.at[pl.ds(off, CE)], dst_v)
        pltpu.sync_copy(data_hbm.at[c, pl.ds(off, CE), :], row_v)
        pltpu.sync_copy(row_v, acc_sh.at[dst_v], add=True)
        return 0

    lax.fori_loop(0, NCH_S, chunk, 0)
    plsc.subcore_barrier()

    def wc(i, _):
        r = rb + i * CE
        pltpu.sync_copy(acc_sh.at[pl.ds(r, CE), :], row_v)
        pltpu.sync_copy(row_v, acc_out.at[c, pl.ds(r, CE), :])
        return 0

    lax.fori_loop(0, RPP // CE, wc, 0)


def _epi_body(acc_ref, den_ref, s_ref, h_ref):
    d = jnp.concatenate([den_ref[0, :, :], den_ref[1, :, :]], axis=1)
    acc = jnp.concatenate([acc_ref[0, :, :], acc_ref[1, :, :]], axis=1)
    msgsum = jnp.where(d > 0.0, acc / jnp.where(d > 0.0, d, 1.0), 0.0)
    h_ref[...] = jnp.maximum(msgsum + s_ref[...], 0.0)


def _epilogue(acc, den, skip):
    return pl.pallas_call(
        _epi_body,
        grid=(N // 400,),
        in_specs=[
            pl.BlockSpec((NC, 400, 128), lambda i: (0, i, 0)),
            pl.BlockSpec((NC, 400, 128), lambda i: (0, i, 0)),
            pl.BlockSpec((400, HID), lambda i: (i, 0)),
        ],
        out_specs=pl.BlockSpec((400, HID), lambda i: (i, 0)),
        out_shape=jax.ShapeDtypeStruct((N, HID), jnp.float32),
    )(acc, den, skip)


@functools.partial(
    pl.kernel,
    mesh=_mesh,
    out_type=[
        jax.ShapeDtypeStruct((NC, 17, 128), jnp.float32),
        jax.ShapeDtypeStruct((17, 128), jnp.float32),
    ],
    scratch_types=[
        pltpu.VMEM((40,), jnp.int32),
        pltpu.VMEM((40, 128), jnp.float32),
        pltpu.VMEM((40, 128), jnp.float32),
        pltpu.VMEM((17, 128), jnp.float32),
        pltpu.VMEM((17, 128), jnp.float32),
        pltpu.VMEM_SHARED((17, 128), jnp.float32),
        pltpu.VMEM_SHARED((17, 128), jnp.float32),
    ],
)
def _pool(h_hbm, batch_hbm, ones_hbm, z17_hbm, z17d_hbm,
          sums_out, cnt_out, bidx_v, rows_v, ones_v, st_v, std_v, acc_sh, cnt_sh):
    c = lax.axis_index("c")
    s = lax.axis_index("s")

    @pl.when(s == 0)
    def _():
        pltpu.sync_copy(z17_hbm, st_v)
        pltpu.sync_copy(st_v, acc_sh)
        pltpu.sync_copy(z17d_hbm, std_v)
        pltpu.sync_copy(std_v, cnt_sh)

    plsc.subcore_barrier()
    pltpu.sync_copy(ones_hbm, ones_v)

    def pchunk(i, _):
        base = s * RPP + i * 40
        pltpu.sync_copy(batch_hbm.at[pl.ds(base, 40)], bidx_v)
        pltpu.sync_copy(h_hbm.at[pl.ds(base, 40), pl.ds(c * 128, 128)], rows_v)
        pltpu.sync_copy(rows_v, acc_sh.at[bidx_v], add=True)
        pltpu.sync_copy(ones_v, cnt_sh.at[bidx_v], add=True)
        return 0

    lax.fori_loop(0, RPP // 40, pchunk, 0)
    plsc.subcore_barrier()

    @pl.when(s == 0)
    def _():
        pltpu.sync_copy(acc_sh, st_v)
        pltpu.sync_copy(st_v, sums_out.at[c])

    @pl.when((s == 0) & (c == 0))
    def _():
        pltpu.sync_copy(cnt_sh, std_v)
        pltpu.sync_copy(std_v, cnt_out)


def _head_body(sums_ref, cnt_ref, gf_ref,
               wg1, bg1, wg2, bg2, wg3, bg3,
               w1a, w1b, b1, w2, b2, w3, b3, out_ref):
    sums = jnp.concatenate([sums_ref[0, :, :], sums_ref[1, :, :]], axis=1)
    sums = sums[0:16, :]
    cnt = cnt_ref[0:16, 0:1]
    xp = sums / jnp.maximum(cnt, 1.0)
    g = gf_ref[...]
    g = jnp.maximum(jnp.dot(g, wg1[...], preferred_element_type=jnp.float32) + bg1[...], 0.0)
    g = jnp.maximum(jnp.dot(g, wg2[...], preferred_element_type=jnp.float32) + bg2[...], 0.0)
    g = jnp.maximum(jnp.dot(g, wg3[...], preferred_element_type=jnp.float32) + bg3[...], 0.0)
    r = (jnp.dot(xp, w1a[...], preferred_element_type=jnp.float32)
         + jnp.dot(g, w1b[...], preferred_element_type=jnp.float32) + b1[...])
    r = jnp.maximum(r, 0.0)
    r = jnp.maximum(jnp.dot(r, w2[...], preferred_element_type=jnp.float32) + b2[...], 0.0)
    out_ref[...] = jnp.dot(r, w3[...], preferred_element_type=jnp.float32) + b3[...]


def _head(sums, cnt, gf, gmlp_params, reg_params):
    (wg1, bg1), (wg2, bg2), (wg3, bg3) = gmlp_params
    (w1, b1), (w2, b2), (w3, b3) = reg_params
    args = [sums, cnt, gf,
            wg1, bg1[None, :], wg2, bg2[None, :], wg3, bg3[None, :],
            w1[:HID], w1[HID:], b1[None, :], w2, b2[None, :], w3, b3[None, :]]
    return pl.pallas_call(
        _head_body,
        out_shape=jax.ShapeDtypeStruct((16, 1), jnp.float32),
    )(*args)


def kernel(x, edge_index, batch, global_features, conv_params, gmlp_params, reg_params):
    src = edge_index[0]
    dst = edge_index[1]
    zrow = jnp.zeros((CE, 128), jnp.float32)

    h = x.astype(jnp.float32)
    for p in conv_params:
        W4 = jnp.concatenate([p['Wq'], p['Wk'], p['Wv'], p['Wskip']], axis=1)
        b4 = jnp.concatenate([p['bq'], p['bk'], p['bv'], p['bskip']])[None, :]
        q, k, v, skip = _qkvs(h, W4, b4)
        qd, ks, vs = _gather3(q, k, v, dst, src)
        alpha, gmax = _alpha(qd, ks)
        ex, msg = _msg(alpha, gmax, vs)
        acc = _scat128(msg, dst, zrow)
        den = _scat128(ex, dst, zrow)
        h = _epilogue(acc, den, skip)

    hp = jnp.concatenate([h, jnp.zeros((NPAD - N, HID), jnp.float32)], axis=0)
    bp = jnp.concatenate([batch, jnp.full((NPAD - N,), 16, jnp.int32)], axis=0)
    ones = jnp.ones((40, 128), jnp.float32)
    z17 = jnp.zeros((17, 128), jnp.float32)
    z17d = jnp.zeros((17, 128), jnp.float32)
    sums, cnt = _pool(hp, bp, ones, z17, z17d)
    out = _head(sums, cnt, global_features, gmlp_params, reg_params)
    return out.reshape(-1)
